# Initial kernel scaffold; baseline (speedup 1.0000x reference)
import jax
import jax.numpy as jnp
from jax.experimental import pallas as pl


def kernel(ngram_ids, ngram_lengths, embedding):
    raise NotImplementedError("write your pallas kernel here")


# speedup vs baseline: 11.9488x; 11.9488x over previous
import functools

import jax
import jax.numpy as jnp
from jax import lax
from jax.experimental import pallas as pl
from jax.experimental.pallas import tpu as pltpu
from jax.experimental.pallas import tpu_sc as plsc

B = 16384
L = 200
D = 32
NC = 2
NS = 16
NW = NC * NS
G = B // NW
ROWS_PER_TILE = G * L
CHUNK = 1024
NCHUNK = ROWS_PER_TILE // CHUNK
TRASH = NS * G


def _body(ids_hbm, len_hbm, tab_hbm, out_hbm,
          acc_sp, ids_v, seg_v, rows_v, len_v, inv_v, obuf, semg, sems):
    c = lax.axis_index("c")
    s = lax.axis_index("s")
    wid = c * NS + s
    base_b = wid * G
    slot0 = s * G

    pltpu.sync_copy(len_hbm.at[pl.ds(base_b, G)], len_v)
    for k in range(G // 16):
        lv = len_v[pl.ds(k * 16, 16)]
        inv_v[pl.ds(k * 16, 16)] = 1.0 / lv.astype(jnp.float32)

    zero = jnp.zeros((16,), jnp.float32)
    for r in range(128):
        for h in range(D // 16):
            obuf[r, pl.ds(h * 16, 16)] = zero
    for p in range(G // 128):
        pltpu.sync_copy(obuf, acc_sp.at[pl.ds(slot0 + p * 128, 128)])

    iota = lax.iota(jnp.int32, 16)

    @pl.loop(0, NCHUNK)
    def _chunk(g):
        idrow = wid * (ROWS_PER_TILE // 128) + g * (CHUNK // 128)
        pltpu.sync_copy(ids_hbm.at[pl.ds(idrow, CHUNK // 128)], ids_v)

        cps = [
            pltpu.async_copy(tab_hbm.at[ids_v.at[j]],
                             rows_v.at[pl.ds(j * 128, 128)], semg)
            for j in range(CHUNK // 128)
        ]

        row0 = wid * ROWS_PER_TILE + g * CHUNK
        for j in range(CHUNK // 128):
            for k in range(8):
                gidx = row0 + (j * 8 + k) * 16 + iota
                q = lax.div(gidx, L)
                l = gidx - q * L
                bl = q - base_b
                lens = plsc.load_gather(len_v, [bl])
                seg = jnp.where(l < lens, bl + slot0, TRASH + s)
                seg_v[j, pl.ds(k * 16, 16)] = seg

        for cp in cps:
            cp.wait()

        cps2 = [
            pltpu.async_copy(rows_v.at[pl.ds(j * 128, 128)],
                             acc_sp.at[seg_v.at[j]], sems, add=True)
            for j in range(CHUNK // 128)
        ]
        for cp in cps2:
            cp.wait()

    for p in range(G // 128):
        pltpu.sync_copy(acc_sp.at[pl.ds(slot0 + p * 128, 128)], obuf)

        @pl.loop(0, 128)
        def _scale(b):
            inv = plsc.load_gather(inv_v, [jnp.broadcast_to(p * 128 + b, (16,))])
            for h in range(D // 16):
                obuf[b, pl.ds(h * 16, 16)] = obuf[b, pl.ds(h * 16, 16)] * inv

        pltpu.sync_copy(obuf, out_hbm.at[pl.ds(base_b + p * 128, 128)])


_bag = pl.kernel(
    _body,
    out_type=jax.ShapeDtypeStruct((B, D), jnp.float32),
    mesh=plsc.VectorSubcoreMesh(core_axis_name="c", subcore_axis_name="s"),
    compiler_params=pltpu.CompilerParams(
        needs_layout_passes=False, use_tc_tiling_on_sc=False),
    scratch_types=[
        pltpu.VMEM_SHARED((NS * G + NS, D), jnp.float32),
        pltpu.VMEM((CHUNK // 128, 128), jnp.int32),
        pltpu.VMEM((CHUNK // 128, 128), jnp.int32),
        pltpu.VMEM((CHUNK, D), jnp.float32),
        pltpu.VMEM((G,), jnp.int32),
        pltpu.VMEM((G,), jnp.float32),
        pltpu.VMEM((128, D), jnp.float32),
        pltpu.SemaphoreType.DMA,
        pltpu.SemaphoreType.DMA,
    ],
)


@jax.jit
def kernel(ngram_ids, ngram_lengths, embedding):
    ids2d = ngram_ids.reshape(B * L // 128, 128)
    return _bag(ids2d, ngram_lengths, embedding)


# speedup vs baseline: 13.6104x; 1.1391x over previous
import functools

import jax
import jax.numpy as jnp
from jax import lax
from jax.experimental import pallas as pl
from jax.experimental.pallas import tpu as pltpu
from jax.experimental.pallas import tpu_sc as plsc

B = 16384
L = 200
D = 32
NC = 2
NS = 16
NW = NC * NS
G = B // NW
ROWS_PER_TILE = G * L
CHUNK = 1024
NCHUNK = ROWS_PER_TILE // CHUNK
TRASH = NS * G


def _body(ids_hbm, len_hbm, tab_hbm, out_hbm,
          acc_sp, ids_v, seg_v, rows_v, len_v, inv_v, obuf,
          semg, sems0, sems1):
    c = lax.axis_index("c")
    s = lax.axis_index("s")
    wid = c * NS + s
    base_b = wid * G
    slot0 = s * G
    sems = (sems0, sems1)
    NDMA = CHUNK // 128

    pltpu.sync_copy(len_hbm.at[pl.ds(base_b, G)], len_v)
    for k in range(G // 16):
        lv = len_v[pl.ds(k * 16, 16)]
        inv_v[pl.ds(k * 16, 16)] = 1.0 / lv.astype(jnp.float32)

    zero = jnp.zeros((16,), jnp.float32)
    for r in range(128):
        for h in range(D // 16):
            obuf[r, pl.ds(h * 16, 16)] = zero
    for p in range(G // 128):
        pltpu.sync_copy(obuf, acc_sp.at[pl.ds(slot0 + p * 128, 128)])

    iota = lax.iota(jnp.int32, 16)

    def stage_ids(g, sub):
        idrow = wid * (ROWS_PER_TILE // 128) + g * NDMA
        pltpu.sync_copy(ids_hbm.at[pl.ds(idrow, NDMA)], ids_v.at[sub])

    def fire_gathers(sub):
        for j in range(NDMA):
            pltpu.async_copy(tab_hbm.at[ids_v.at[sub].at[j]],
                             rows_v.at[sub].at[pl.ds(j * 128, 128)], semg)

    def drain(sem, sub):
        pltpu.make_async_copy(tab_hbm.at[pl.ds(0, CHUNK)],
                              rows_v.at[sub], sem).wait()

    def compute_seg(g, sub):
        row0 = wid * ROWS_PER_TILE + g * CHUNK
        for j in range(NDMA):
            for k in range(8):
                gidx = row0 + (j * 8 + k) * 16 + iota
                q = lax.div(gidx, L)
                l = gidx - q * L
                bl = q - base_b
                lens = plsc.load_gather(len_v, [bl])
                seg = jnp.where(l < lens, bl + slot0, TRASH + s)
                seg_v[sub, j, pl.ds(k * 16, 16)] = seg

    def fire_scatters(sub):
        for j in range(NDMA):
            pltpu.async_copy(rows_v.at[sub].at[pl.ds(j * 128, 128)],
                             acc_sp.at[seg_v.at[sub].at[j]], sems[sub],
                             add=True)

    stage_ids(0, 0)
    fire_gathers(0)

    @pl.loop(0, NCHUNK, step=2)
    def _chunk(go):
        for sub in range(2):
            g = go + sub
            compute_seg(g, sub)
            drain(semg, sub)
            fire_scatters(sub)

            @pl.when(g < NCHUNK - 1)
            def _prep():
                @pl.when(g >= 1)
                def _free():
                    drain(sems[1 - sub], 1 - sub)
                stage_ids(g + 1, 1 - sub)
                fire_gathers(1 - sub)

    drain(sems[0], 0)
    drain(sems[1], 1)

    for p in range(G // 128):
        pltpu.sync_copy(acc_sp.at[pl.ds(slot0 + p * 128, 128)], obuf)

        @pl.loop(0, 128)
        def _scale(b):
            inv = plsc.load_gather(inv_v, [jnp.broadcast_to(p * 128 + b, (16,))])
            for h in range(D // 16):
                obuf[b, pl.ds(h * 16, 16)] = obuf[b, pl.ds(h * 16, 16)] * inv

        pltpu.sync_copy(obuf, out_hbm.at[pl.ds(base_b + p * 128, 128)])


_bag = pl.kernel(
    _body,
    out_type=jax.ShapeDtypeStruct((B, D), jnp.float32),
    mesh=plsc.VectorSubcoreMesh(core_axis_name="c", subcore_axis_name="s"),
    compiler_params=pltpu.CompilerParams(
        needs_layout_passes=False, use_tc_tiling_on_sc=False),
    scratch_types=[
        pltpu.VMEM_SHARED((NS * G + NS, D), jnp.float32),
        pltpu.VMEM((2, CHUNK // 128, 128), jnp.int32),
        pltpu.VMEM((2, CHUNK // 128, 128), jnp.int32),
        pltpu.VMEM((2, CHUNK, D), jnp.float32),
        pltpu.VMEM((G,), jnp.int32),
        pltpu.VMEM((G,), jnp.float32),
        pltpu.VMEM((128, D), jnp.float32),
        pltpu.SemaphoreType.DMA,
        pltpu.SemaphoreType.DMA,
        pltpu.SemaphoreType.DMA,
    ],
)


@jax.jit
def kernel(ngram_ids, ngram_lengths, embedding):
    ids2d = ngram_ids.reshape(B * L // 128, 128)
    return _bag(ids2d, ngram_lengths, embedding)
